# baseline (speedup 1.0000x reference)
import jax
import jax.numpy as jnp
from jax import lax
from jax.experimental import pallas as pl
from jax.experimental.pallas import tpu as pltpu
from jax.experimental.pallas import tpu_sc as plsc

N, D, K, M, S = 10000, 256, 8, 1000, 1250
NEDGE = 125
THPOS = 12
EPS_PD = 1e-6
EPS_COS = 1e-8

NW = 32
BPAD = 1024
GPT = BPAD // NW
FW = 5
RPT = 2000
MPAD = 1024

NB = 5
BLK = N // NB


def _sc_body(node_hbm, idx_hbm, mask_hbm, wval_hbm, f_hbm, wts_hbm,
             idx_v, rows_v, mask_v, flag_v, wval_v,
             sem_i, sem_m, sem_w, sem_g):
    c = lax.axis_index("c")
    s = lax.axis_index("s")
    wid = s * 2 + c

    gbase = wid * GPT
    pltpu.async_copy(idx_hbm.at[pl.ds(gbase, GPT)], idx_v, sem_i).wait()
    gath = pltpu.async_copy(node_hbm.at[idx_v], rows_v, sem_g)

    @pl.when(wid < FW)
    def _flags():
        fbase = wid * RPT
        cm = pltpu.async_copy(mask_hbm, mask_v, sem_m)
        cw = pltpu.async_copy(wval_hbm, wval_v, sem_w)
        ones16 = jnp.ones((16,), jnp.float32)
        for i in range(RPT // 16):
            flag_v[pl.ds(i * 16, 16)] = ones16
        cm.wait()
        cw.wait()
        wv = wval_v[...]
        for i in range(MPAD // 16):
            mv = mask_v[pl.ds(i * 16, 16)]
            off = mv - fbase
            valid = (off >= 0) & (off < RPT)
            offc = jnp.minimum(jnp.maximum(off, 0), RPT - 1)
            plsc.store_scatter(flag_v, [offc], wv, mask=valid)
        pltpu.sync_copy(flag_v, wts_hbm.at[wid, 0, pl.ds(0, RPT)])

    gath.wait()
    pltpu.sync_copy(rows_v, f_hbm.at[pl.ds(gbase, GPT)])


NBUF = 3


def _tc_body(nf_any, lab_ref, cen_ref, wts_ref, f_any, out_ref,
             dscr, f_vmem, xbuf, fsem, xsem):
    b = pl.program_id(0)

    @pl.when(b == 0)
    def _start_f():
        pltpu.make_async_copy(f_any, f_vmem, fsem).start()
        for k in range(NBUF - 1):
            pltpu.make_async_copy(nf_any.at[pl.ds(k * BLK, BLK), :],
                                  xbuf.at[k], xsem.at[k]).start()

    nxt = b + NBUF - 1

    @pl.when(nxt < NB)
    def _start_next():
        pltpu.make_async_copy(nf_any.at[pl.ds(nxt * BLK, BLK), :],
                              xbuf.at[nxt % NBUF], xsem.at[nxt % NBUF]).start()

    bm = b % NBUF
    pltpu.make_async_copy(nf_any.at[pl.ds(b * BLK, BLK), :],
                          xbuf.at[bm], xsem.at[bm]).wait()

    x = xbuf[pl.ds(bm, 1), :, :].reshape(BLK, D)
    lab = lab_ref[0, 0, :]
    cen = cen_ref[...]
    oh = (lab[:, None] == lax.broadcasted_iota(jnp.int32, (BLK, K), 1)
          ).astype(jnp.float32)
    csel = lax.dot_general(oh, cen, (((1,), (0,)), ((), ())),
                           preferred_element_type=jnp.float32)
    diff = x - csel + EPS_PD
    d = jnp.sqrt(jnp.sum(diff * diff, axis=1))
    dscr[pl.ds(b, 1), :, :] = d.reshape(1, 1, BLK)

    @pl.when(b == NB - 1)
    def _fin():
        inner = jnp.sum(dscr[...].reshape(NB, BLK) * wts_ref[...].reshape(NB, BLK))

        pltpu.make_async_copy(f_any, f_vmem, fsem).wait()
        F = f_vmem[0:K * NEDGE, :].reshape(K, NEDGE, D)
        fn = jnp.maximum(jnp.sqrt(jnp.sum(F * F, axis=2)), EPS_COS)
        cd = cen[:, None, :] - cen[None, :, :]
        tn = jnp.maximum(jnp.sqrt(jnp.sum(cd * cd, axis=2)), EPS_COS)

        means = []
        cnts = []
        for i in range(K):
            Fi = F[i]
            g = lax.dot_general(cen, Fi, (((1,), (1,)), ((), ())),
                                preferred_element_type=jnp.float32)
            num = g[i, :][None, :] - g
            cos = num / (tn[i, :][:, None] * fn[i, :][None, :])
            less = (cos[:, None, :] < cos[:, :, None]).astype(jnp.float32)
            cnt_less = jnp.sum(less, axis=2)
            keep = (cnt_less > jnp.float32(THPOS)).astype(jnp.float32)
            cnt = jnp.sum(keep, axis=1)
            ssum = lax.dot_general(keep, Fi, (((1,), (0,)), ((), ())),
                                   preferred_element_type=jnp.float32)
            mean = ssum / jnp.maximum(cnt, 1.0)[:, None]
            means.append(mean)
            cnts.append(cnt)

        L2 = jnp.float32(0.0)
        for i in range(K):
            for j in range(i + 1, K):
                dd = means[i][j] - means[j][i] + EPS_PD
                dist = jnp.sqrt(jnp.sum(dd * dd))
                ok = (cnts[i][j] > 0.0) & (cnts[j][i] > 0.0)
                L2 = L2 + jnp.where(ok, dist, jnp.float32(0.0))

        out_ref[...] = jnp.broadcast_to(inner - L2, (1, 1))


def kernel(node_fea, clu_label, center_fea, mask_nodes, mask_weight, sort_idx_rst):
    node_fea = node_fea.astype(jnp.float32)
    center_fea = center_fea.astype(jnp.float32)

    idx = sort_idx_rst[:, S - NEDGE:].astype(jnp.int32).reshape(-1)
    idx = jnp.concatenate([idx, jnp.zeros((BPAD - K * NEDGE,), jnp.int32)])
    mask = jnp.concatenate([mask_nodes.astype(jnp.int32),
                            jnp.full((MPAD - M,), -1, jnp.int32)])
    mwf = jnp.asarray(mask_weight, jnp.float32)
    wval = jnp.full((16,), 2.0, jnp.float32) + mwf

    sc_call = pl.kernel(
        _sc_body,
        out_type=[jax.ShapeDtypeStruct((BPAD, D), jnp.float32),
                  jax.ShapeDtypeStruct((NB, 1, BLK), jnp.float32)],
        mesh=plsc.VectorSubcoreMesh(core_axis_name="c", subcore_axis_name="s"),
        compiler_params=pltpu.CompilerParams(needs_layout_passes=False),
        scratch_types=[
            pltpu.VMEM((GPT,), jnp.int32),
            pltpu.VMEM((GPT, D), jnp.float32),
            pltpu.VMEM((MPAD,), jnp.int32),
            pltpu.VMEM((RPT,), jnp.float32),
            pltpu.VMEM((16,), jnp.float32),
            pltpu.SemaphoreType.DMA,
            pltpu.SemaphoreType.DMA,
            pltpu.SemaphoreType.DMA,
            pltpu.SemaphoreType.DMA,
        ],
    )
    F, wts = sc_call(node_fea, idx, mask, wval)

    labs = clu_label.astype(jnp.int32).reshape(NB, 1, BLK)

    out = pl.pallas_call(
        _tc_body,
        grid=(NB,),
        in_specs=[
            pl.BlockSpec(memory_space=pl.ANY),
            pl.BlockSpec((1, 1, BLK), lambda b: (b, 0, 0)),
            pl.BlockSpec((K, D), lambda b: (0, 0)),
            pl.BlockSpec((NB, 1, BLK), lambda b: (0, 0, 0)),
            pl.BlockSpec(memory_space=pl.ANY),
        ],
        out_specs=pl.BlockSpec((1, 1), lambda b: (0, 0)),
        out_shape=jax.ShapeDtypeStruct((1, 1), jnp.float32),
        scratch_shapes=[
            pltpu.VMEM((NB, 1, BLK), jnp.float32),
            pltpu.VMEM((BPAD, D), jnp.float32),
            pltpu.VMEM((NBUF, BLK, D), jnp.float32),
            pltpu.SemaphoreType.DMA,
            pltpu.SemaphoreType.DMA((NBUF,)),
        ],
    )(node_fea, labs, center_fea, wts, F)
    return out.reshape(1)


# speedup vs baseline: 1.0533x; 1.0533x over previous
import jax
import jax.numpy as jnp
from jax import lax
from jax.experimental import pallas as pl
from jax.experimental.pallas import tpu as pltpu
from jax.experimental.pallas import tpu_sc as plsc

N, D, K, M, S = 10000, 256, 8, 1000, 1250
NEDGE = 125
THPOS = 12
EPS_PD = 1e-6
EPS_COS = 1e-8

NW = 32
BPAD = 1024
GPT = BPAD // NW
FW = 5
RPT = 2000
MPAD = 1024

NB = 5
BLK = N // NB


def _sc_body(node_hbm, idx_hbm, mask_hbm, wval_hbm, f_hbm, wts_hbm,
             idx_v, rows_v, mask_v, flag_v, wval_v,
             sem_i, sem_m, sem_w, sem_g):
    c = lax.axis_index("c")
    s = lax.axis_index("s")
    wid = s * 2 + c

    gbase = wid * GPT
    pltpu.async_copy(idx_hbm.at[pl.ds(gbase, GPT)], idx_v, sem_i).wait()
    gath = pltpu.async_copy(node_hbm.at[idx_v], rows_v, sem_g)

    @pl.when(wid < FW)
    def _flags():
        fbase = wid * RPT
        cm = pltpu.async_copy(mask_hbm, mask_v, sem_m)
        cw = pltpu.async_copy(wval_hbm, wval_v, sem_w)
        ones16 = jnp.ones((16,), jnp.float32)
        for i in range(RPT // 16):
            flag_v[pl.ds(i * 16, 16)] = ones16
        cm.wait()
        cw.wait()
        wv = wval_v[...]
        for i in range(MPAD // 16):
            mv = mask_v[pl.ds(i * 16, 16)]
            off = mv - fbase
            valid = (off >= 0) & (off < RPT)
            offc = jnp.minimum(jnp.maximum(off, 0), RPT - 1)
            plsc.store_scatter(flag_v, [offc], wv, mask=valid)
        pltpu.sync_copy(flag_v, wts_hbm.at[wid, 0, pl.ds(0, RPT)])

    gath.wait()
    pltpu.sync_copy(rows_v, f_hbm.at[pl.ds(gbase, GPT)])


def _tc_body(nf_ref, lab_ref, cen_ref, wts_ref, f_any, out_ref,
             dscr, f_vmem, fsem):
    b = pl.program_id(0)

    @pl.when(b == 0)
    def _start_f():
        pltpu.make_async_copy(f_any, f_vmem, fsem).start()

    x = nf_ref[...]
    lab = lab_ref[0, 0, :]
    cen = cen_ref[...]
    oh = (lab[:, None] == lax.broadcasted_iota(jnp.int32, (BLK, K), 1)
          ).astype(jnp.float32)
    csel = lax.dot_general(oh, cen, (((1,), (0,)), ((), ())),
                           preferred_element_type=jnp.float32)
    diff = x - csel + EPS_PD
    d = jnp.sqrt(jnp.sum(diff * diff, axis=1))
    dscr[pl.ds(b, 1), :, :] = d.reshape(1, 1, BLK)

    @pl.when(b == NB - 1)
    def _fin():
        inner = jnp.sum(dscr[...].reshape(NB, BLK) * wts_ref[...].reshape(NB, BLK))

        pltpu.make_async_copy(f_any, f_vmem, fsem).wait()
        F = f_vmem[0:K * NEDGE, :].reshape(K, NEDGE, D)
        fn = jnp.maximum(jnp.sqrt(jnp.sum(F * F, axis=2)), EPS_COS)
        cd = cen[:, None, :] - cen[None, :, :]
        tn = jnp.maximum(jnp.sqrt(jnp.sum(cd * cd, axis=2)), EPS_COS)

        cos_rows = []
        for i in range(K):
            g = lax.dot_general(cen, F[i], (((1,), (1,)), ((), ())),
                                preferred_element_type=jnp.float32)
            num = g[i, :][None, :] - g
            cos = num / (tn[i, :][:, None] * fn[i, :][None, :])
            cos_rows.append(cos.reshape(1, K, NEDGE))
        cos_all = jnp.concatenate(cos_rows, axis=0)

        bits = lax.bitcast_convert_type(cos_all, jnp.int32)
        xm = lax.shift_right_arithmetic(bits, 31) | jnp.int32(-2147483648)
        mu = lax.bitcast_convert_type(bits ^ xm, jnp.uint32)
        r_u = jnp.zeros((K, K, 1), jnp.uint32)
        for bit in range(31, -1, -1):
            cand = r_u | jnp.uint32(1 << bit)
            cless = jnp.sum((mu < cand).astype(jnp.float32), axis=2,
                            keepdims=True)
            r_u = jnp.where(cless <= jnp.float32(THPOS), cand, r_u)
        keep_all = (mu > r_u).astype(jnp.float32)
        cnt_all = jnp.sum(keep_all, axis=2)

        means = []
        cnts = []
        for i in range(K):
            ssum = lax.dot_general(keep_all[i], F[i], (((1,), (0,)), ((), ())),
                                   preferred_element_type=jnp.float32)
            mean = ssum / jnp.maximum(cnt_all[i], 1.0)[:, None]
            means.append(mean)
            cnts.append(cnt_all[i])

        L2 = jnp.float32(0.0)
        for i in range(K):
            for j in range(i + 1, K):
                dd = means[i][j] - means[j][i] + EPS_PD
                dist = jnp.sqrt(jnp.sum(dd * dd))
                ok = (cnts[i][j] > 0.0) & (cnts[j][i] > 0.0)
                L2 = L2 + jnp.where(ok, dist, jnp.float32(0.0))

        out_ref[...] = jnp.broadcast_to(inner - L2, (1, 1))


def kernel(node_fea, clu_label, center_fea, mask_nodes, mask_weight, sort_idx_rst):
    node_fea = node_fea.astype(jnp.float32)
    center_fea = center_fea.astype(jnp.float32)

    idx = sort_idx_rst[:, S - NEDGE:].astype(jnp.int32).reshape(-1)
    idx = jnp.concatenate([idx, jnp.zeros((BPAD - K * NEDGE,), jnp.int32)])
    mask = jnp.concatenate([mask_nodes.astype(jnp.int32),
                            jnp.full((MPAD - M,), -1, jnp.int32)])
    mwf = jnp.asarray(mask_weight, jnp.float32)
    wval = jnp.full((16,), 2.0, jnp.float32) + mwf

    sc_call = pl.kernel(
        _sc_body,
        out_type=[jax.ShapeDtypeStruct((BPAD, D), jnp.float32),
                  jax.ShapeDtypeStruct((NB, 1, BLK), jnp.float32)],
        mesh=plsc.VectorSubcoreMesh(core_axis_name="c", subcore_axis_name="s"),
        compiler_params=pltpu.CompilerParams(needs_layout_passes=False),
        scratch_types=[
            pltpu.VMEM((GPT,), jnp.int32),
            pltpu.VMEM((GPT, D), jnp.float32),
            pltpu.VMEM((MPAD,), jnp.int32),
            pltpu.VMEM((RPT,), jnp.float32),
            pltpu.VMEM((16,), jnp.float32),
            pltpu.SemaphoreType.DMA,
            pltpu.SemaphoreType.DMA,
            pltpu.SemaphoreType.DMA,
            pltpu.SemaphoreType.DMA,
        ],
    )
    F, wts = sc_call(node_fea, idx, mask, wval)

    labs = clu_label.astype(jnp.int32).reshape(NB, 1, BLK)

    out = pl.pallas_call(
        _tc_body,
        grid=(NB,),
        in_specs=[
            pl.BlockSpec((BLK, D), lambda b: (b, 0)),
            pl.BlockSpec((1, 1, BLK), lambda b: (b, 0, 0)),
            pl.BlockSpec((K, D), lambda b: (0, 0)),
            pl.BlockSpec((NB, 1, BLK), lambda b: (0, 0, 0)),
            pl.BlockSpec(memory_space=pl.ANY),
        ],
        out_specs=pl.BlockSpec((1, 1), lambda b: (0, 0)),
        out_shape=jax.ShapeDtypeStruct((1, 1), jnp.float32),
        scratch_shapes=[
            pltpu.VMEM((NB, 1, BLK), jnp.float32),
            pltpu.VMEM((BPAD, D), jnp.float32),
            pltpu.SemaphoreType.DMA,
        ],
    )(node_fea, labs, center_fea, wts, F)
    return out.reshape(1)


# speedup vs baseline: 1.0999x; 1.0443x over previous
import jax
import jax.numpy as jnp
from jax import lax
from jax.experimental import pallas as pl
from jax.experimental.pallas import tpu as pltpu
from jax.experimental.pallas import tpu_sc as plsc

N, D, K, M, S = 10000, 256, 8, 1000, 1250
NEDGE = 125
THPOS = 12
EPS_PD = 1e-6
EPS_COS = 1e-8

NW = 32
NG = K * NEDGE
GPT = 32
FW = 5
RPT = 2000

NB = 5
BLK = N // NB


def _sc_body(node_hbm, idx_hbm, mask_hbm, wval_hbm, f_hbm, wts_hbm,
             idx_v, rows_v, mask_v, flag_v, wval_v,
             sem_i, sem_m, sem_w, sem_g):
    c = lax.axis_index("c")
    s = lax.axis_index("s")
    wid = s * 2 + c

    gbase = wid * GPT
    last = wid == NW - 1

    @pl.when(jnp.logical_not(last))
    def _g_full():
        pltpu.async_copy(idx_hbm.at[pl.ds(gbase, GPT)], idx_v, sem_i).wait()
        pltpu.make_async_copy(node_hbm.at[idx_v], rows_v, sem_g).start()

    @pl.when(last)
    def _g_tail():
        pltpu.async_copy(idx_hbm.at[pl.ds(NG - 8, 8)],
                         idx_v.at[pl.ds(0, 8)], sem_i).wait()
        pltpu.make_async_copy(node_hbm.at[idx_v.at[pl.ds(0, 8)]],
                              rows_v.at[pl.ds(0, 8), :], sem_g).start()

    @pl.when(wid < FW)
    def _flags():
        fbase = wid * RPT
        cm = pltpu.async_copy(mask_hbm, mask_v, sem_m)
        cw = pltpu.async_copy(wval_hbm, wval_v, sem_w)
        ones16 = jnp.ones((16,), jnp.float32)
        for i in range(RPT // 16):
            flag_v[pl.ds(i * 16, 16)] = ones16
        cm.wait()
        cw.wait()
        wv = wval_v[...]
        offsets = [i * 16 for i in range(M // 16)] + [M - 16]
        for o in offsets:
            mv = mask_v[pl.ds(o, 16)]
            off = mv - fbase
            valid = (off >= 0) & (off < RPT)
            offc = jnp.minimum(jnp.maximum(off, 0), RPT - 1)
            plsc.store_scatter(flag_v, [offc], wv, mask=valid)
        pltpu.sync_copy(flag_v, wts_hbm.at[wid, 0, pl.ds(0, RPT)])

    @pl.when(jnp.logical_not(last))
    def _w_full():
        pltpu.make_async_copy(node_hbm.at[idx_v], rows_v, sem_g).wait()
        pltpu.sync_copy(rows_v, f_hbm.at[pl.ds(gbase, GPT)])

    @pl.when(last)
    def _w_tail():
        pltpu.make_async_copy(node_hbm.at[idx_v.at[pl.ds(0, 8)]],
                              rows_v.at[pl.ds(0, 8), :], sem_g).wait()
        pltpu.sync_copy(rows_v.at[pl.ds(0, 8), :], f_hbm.at[pl.ds(NG - 8, 8)])


def _tc_body(nf_ref, lab_ref, cen_ref, wts_ref, f_any, out_ref,
             dscr, f_vmem, fsem):
    b = pl.program_id(0)

    @pl.when(b == 0)
    def _start_f():
        pltpu.make_async_copy(f_any, f_vmem, fsem).start()

    x = nf_ref[...]
    lab = lab_ref[0, 0, :]
    cen = cen_ref[...]
    oh = (lab[:, None] == lax.broadcasted_iota(jnp.int32, (BLK, K), 1)
          ).astype(jnp.float32)
    csel = lax.dot_general(oh, cen, (((1,), (0,)), ((), ())),
                           preferred_element_type=jnp.float32)
    diff = x - csel + EPS_PD
    d = jnp.sqrt(jnp.sum(diff * diff, axis=1))
    dscr[pl.ds(b, 1), :, :] = d.reshape(1, 1, BLK)

    @pl.when(b == NB - 1)
    def _fin():
        inner = jnp.sum(dscr[...].reshape(NB, BLK) * wts_ref[...].reshape(NB, BLK))

        pltpu.make_async_copy(f_any, f_vmem, fsem).wait()
        F = f_vmem[...].reshape(K, NEDGE, D)
        fn = jnp.maximum(jnp.sqrt(jnp.sum(F * F, axis=2)), EPS_COS)
        cd = cen[:, None, :] - cen[None, :, :]
        tn = jnp.maximum(jnp.sqrt(jnp.sum(cd * cd, axis=2)), EPS_COS)

        cos_rows = []
        for i in range(K):
            g = lax.dot_general(cen, F[i], (((1,), (1,)), ((), ())),
                                preferred_element_type=jnp.float32)
            num = g[i, :][None, :] - g
            cos = num / (tn[i, :][:, None] * fn[i, :][None, :])
            cos_rows.append(cos.reshape(1, K, NEDGE))
        cos_all = jnp.concatenate(cos_rows, axis=0)

        bits = lax.bitcast_convert_type(cos_all, jnp.int32)
        xm = lax.shift_right_arithmetic(bits, 31) | jnp.int32(-2147483648)
        mu = lax.bitcast_convert_type(bits ^ xm, jnp.uint32)
        r_u = jnp.zeros((K, K, 1), jnp.uint32)
        for bit in range(31, -1, -1):
            cand = r_u | jnp.uint32(1 << bit)
            cless = jnp.sum((mu < cand).astype(jnp.float32), axis=2,
                            keepdims=True)
            r_u = jnp.where(cless <= jnp.float32(THPOS), cand, r_u)
        keep_all = (mu > r_u).astype(jnp.float32)
        cnt_all = jnp.sum(keep_all, axis=2)

        means = []
        cnts = []
        for i in range(K):
            ssum = lax.dot_general(keep_all[i], F[i], (((1,), (0,)), ((), ())),
                                   preferred_element_type=jnp.float32)
            mean = ssum / jnp.maximum(cnt_all[i], 1.0)[:, None]
            means.append(mean)
            cnts.append(cnt_all[i])

        L2 = jnp.float32(0.0)
        for i in range(K):
            for j in range(i + 1, K):
                dd = means[i][j] - means[j][i] + EPS_PD
                dist = jnp.sqrt(jnp.sum(dd * dd))
                ok = (cnts[i][j] > 0.0) & (cnts[j][i] > 0.0)
                L2 = L2 + jnp.where(ok, dist, jnp.float32(0.0))

        out_ref[...] = jnp.broadcast_to(inner - L2, (1, 1))


def kernel(node_fea, clu_label, center_fea, mask_nodes, mask_weight, sort_idx_rst):
    node_fea = node_fea.astype(jnp.float32)
    center_fea = center_fea.astype(jnp.float32)

    idx = sort_idx_rst[:, S - NEDGE:].astype(jnp.int32).reshape(-1)
    mask = mask_nodes.astype(jnp.int32)
    mwf = jnp.asarray(mask_weight, jnp.float32)
    wval = jnp.full((16,), 2.0, jnp.float32) + mwf

    sc_call = pl.kernel(
        _sc_body,
        out_type=[jax.ShapeDtypeStruct((NG, D), jnp.float32),
                  jax.ShapeDtypeStruct((NB, 1, BLK), jnp.float32)],
        mesh=plsc.VectorSubcoreMesh(core_axis_name="c", subcore_axis_name="s"),
        compiler_params=pltpu.CompilerParams(needs_layout_passes=False),
        scratch_types=[
            pltpu.VMEM((GPT,), jnp.int32),
            pltpu.VMEM((GPT, D), jnp.float32),
            pltpu.VMEM((M,), jnp.int32),
            pltpu.VMEM((RPT,), jnp.float32),
            pltpu.VMEM((16,), jnp.float32),
            pltpu.SemaphoreType.DMA,
            pltpu.SemaphoreType.DMA,
            pltpu.SemaphoreType.DMA,
            pltpu.SemaphoreType.DMA,
        ],
    )
    F, wts = sc_call(node_fea, idx, mask, wval)

    labs = clu_label.astype(jnp.int32).reshape(NB, 1, BLK)

    out = pl.pallas_call(
        _tc_body,
        grid=(NB,),
        in_specs=[
            pl.BlockSpec((BLK, D), lambda b: (b, 0)),
            pl.BlockSpec((1, 1, BLK), lambda b: (b, 0, 0)),
            pl.BlockSpec((K, D), lambda b: (0, 0)),
            pl.BlockSpec((NB, 1, BLK), lambda b: (0, 0, 0)),
            pl.BlockSpec(memory_space=pl.ANY),
        ],
        out_specs=pl.BlockSpec((1, 1), lambda b: (0, 0)),
        out_shape=jax.ShapeDtypeStruct((1, 1), jnp.float32),
        scratch_shapes=[
            pltpu.VMEM((NB, 1, BLK), jnp.float32),
            pltpu.VMEM((NG, D), jnp.float32),
            pltpu.SemaphoreType.DMA,
        ],
    )(node_fea, labs, center_fea, wts, F)
    return out.reshape(1)
